# baseline (device time: 23399 ns/iter reference)
import jax
import jax.numpy as jnp
from jax import lax
from jax.experimental import pallas as pl
from jax.experimental.pallas import tpu as pltpu

N_DEV = 4
B, SQ, SKV, DH = 2, 128, 128, 64
H_LOC = 4
D_MODEL = 512
D_LOC = H_LOC * DH


def kernel(x, Wq, K_ext, V_ext, Wo):
    my = lax.axis_index("i")
    xf = x.reshape(B * SQ, D_MODEL)
    K2 = lax.dynamic_slice_in_dim(K_ext, my * H_LOC, H_LOC, axis=2)
    K2 = K2.reshape(B * SKV, D_LOC)
    V2 = lax.dynamic_slice_in_dim(V_ext, my * H_LOC, H_LOC, axis=2)
    V2 = V2.reshape(B * SKV, D_LOC)

    def body(x_ref, wq_ref, k_ref, v_ref, wo_ref, out_ref,
             ctx_ref, comm_ref, send_sems, recv_sems):
        my_pos = lax.axis_index("i")
        left = lax.rem(my_pos + N_DEV - 1, N_DEV)
        right = lax.rem(my_pos + 1, N_DEV)

        barrier_sem = pltpu.get_barrier_semaphore()
        for nbr in (left, right):
            pl.semaphore_signal(
                barrier_sem, inc=1,
                device_id=(nbr,), device_id_type=pl.DeviceIdType.MESH,
            )
        pl.semaphore_wait(barrier_sem, 2)

        xb = x_ref[:].astype(jnp.bfloat16)
        wq = wq_ref[:].astype(jnp.bfloat16)
        q = lax.dot(xb, wq, preferred_element_type=jnp.float32)
        qb = (q * 0.125).astype(jnp.bfloat16)
        kb = k_ref[:].astype(jnp.bfloat16)
        vb = v_ref[:].astype(jnp.bfloat16)

        for b in range(B):
            rows = slice(b * SQ, (b + 1) * SQ)
            for h in range(H_LOC):
                cols = slice(h * DH, (h + 1) * DH)
                q_bh = qb[rows, cols]
                k_bh = kb[rows, cols]
                v_bh = vb[rows, cols]
                s = lax.dot_general(
                    q_bh, k_bh, (((1,), (1,)), ((), ())),
                    preferred_element_type=jnp.float32,
                )
                m = jnp.max(s, axis=1, keepdims=True)
                e = jnp.exp(s - m)
                w = e / jnp.sum(e, axis=1, keepdims=True)
                ctx_bh = lax.dot(
                    w.astype(jnp.bfloat16), v_bh,
                    preferred_element_type=jnp.float32,
                )
                ctx_ref[rows, cols] = ctx_bh.astype(jnp.bfloat16)

        wo = wo_ref[:].astype(jnp.bfloat16)
        partial = lax.dot(ctx_ref[:], wo, preferred_element_type=jnp.float32)
        out_ref[:] = partial
        comm_ref[0] = partial.astype(jnp.bfloat16)

        for hop in range(N_DEV - 1):
            send_slot = hop % 2
            recv_slot = (hop + 1) % 2
            rdma = pltpu.make_async_remote_copy(
                src_ref=comm_ref.at[send_slot],
                dst_ref=comm_ref.at[recv_slot],
                send_sem=send_sems.at[send_slot],
                recv_sem=recv_sems.at[recv_slot],
                device_id=(right,),
                device_id_type=pl.DeviceIdType.MESH,
            )
            rdma.start()
            rdma.wait()
            out_ref[:] += comm_ref[recv_slot].astype(jnp.float32)

    out = pl.pallas_call(
        body,
        out_shape=jax.ShapeDtypeStruct((B * SQ, D_MODEL), jnp.float32),
        in_specs=[pl.BlockSpec(memory_space=pltpu.VMEM)] * 5,
        out_specs=pl.BlockSpec(memory_space=pltpu.VMEM),
        scratch_shapes=[
            pltpu.VMEM((B * SQ, D_LOC), jnp.bfloat16),
            pltpu.VMEM((2, B * SQ, D_MODEL), jnp.bfloat16),
            pltpu.SemaphoreType.DMA((2,)),
            pltpu.SemaphoreType.DMA((2,)),
        ],
        compiler_params=pltpu.CompilerParams(collective_id=0),
    )(xf, Wq, K2, V2, Wo)
    return out.reshape(B, SQ, D_MODEL)


# device time: 9461 ns/iter; 2.4732x vs baseline; 2.4732x over previous
import jax
import jax.numpy as jnp
from jax import lax
from jax.experimental import pallas as pl
from jax.experimental.pallas import tpu as pltpu

N_DEV = 4
B, SQ, SKV, DH = 2, 128, 128, 64
H_LOC = 4
D_MODEL = 512
D_LOC = H_LOC * DH


def kernel(x, Wq, K_ext, V_ext, Wo):
    my = lax.axis_index("i")
    xf = x.reshape(B * SQ, D_MODEL)
    K2 = lax.dynamic_slice_in_dim(K_ext, my * H_LOC, H_LOC, axis=2)
    K2 = K2.reshape(B * SKV, D_LOC)
    V2 = lax.dynamic_slice_in_dim(V_ext, my * H_LOC, H_LOC, axis=2)
    V2 = V2.reshape(B * SKV, D_LOC)

    def body(x_ref, wq_ref, k_ref, v_ref, wo_ref, out_ref,
             ctx_ref, send_ref, recv_ref, send_sems, recv_sems):
        my_pos = lax.axis_index("i")
        p1 = my_pos ^ 1
        p2 = (N_DEV - 1) - my_pos

        barrier_sem = pltpu.get_barrier_semaphore()
        for nbr in (p1, p2):
            pl.semaphore_signal(
                barrier_sem, inc=1,
                device_id=(nbr,), device_id_type=pl.DeviceIdType.MESH,
            )
        pl.semaphore_wait(barrier_sem, 2)

        xb = x_ref[:].astype(jnp.bfloat16)
        wq = wq_ref[:].astype(jnp.bfloat16)
        q = lax.dot(xb, wq, preferred_element_type=jnp.float32)
        qb = (q * 0.125).astype(jnp.bfloat16)
        kb = k_ref[:].astype(jnp.bfloat16)
        vb = v_ref[:].astype(jnp.bfloat16)

        for b in range(B):
            rows = slice(b * SQ, (b + 1) * SQ)
            for h in range(H_LOC):
                cols = slice(h * DH, (h + 1) * DH)
                q_bh = qb[rows, cols]
                k_bh = kb[rows, cols]
                v_bh = vb[rows, cols]
                s = lax.dot_general(
                    q_bh, k_bh, (((1,), (1,)), ((), ())),
                    preferred_element_type=jnp.float32,
                )
                m = jnp.max(s, axis=1, keepdims=True)
                e = jnp.exp(s - m)
                w = e / jnp.sum(e, axis=1, keepdims=True)
                ctx_bh = lax.dot(
                    w.astype(jnp.bfloat16), v_bh,
                    preferred_element_type=jnp.float32,
                )
                ctx_ref[rows, cols] = ctx_bh.astype(jnp.bfloat16)

        wo = wo_ref[:].astype(jnp.bfloat16)
        partial = lax.dot(ctx_ref[:], wo, preferred_element_type=jnp.float32)
        out_ref[:] = partial
        send_ref[0] = partial.astype(jnp.bfloat16)

        rdma1 = pltpu.make_async_remote_copy(
            src_ref=send_ref.at[0],
            dst_ref=recv_ref.at[0],
            send_sem=send_sems.at[0],
            recv_sem=recv_sems.at[0],
            device_id=(p1,),
            device_id_type=pl.DeviceIdType.MESH,
        )
        rdma1.start()
        rdma1.wait_recv()
        out_ref[:] += recv_ref[0].astype(jnp.float32)

        send_ref[1] = out_ref[:].astype(jnp.bfloat16)
        rdma2 = pltpu.make_async_remote_copy(
            src_ref=send_ref.at[1],
            dst_ref=recv_ref.at[1],
            send_sem=send_sems.at[1],
            recv_sem=recv_sems.at[1],
            device_id=(p2,),
            device_id_type=pl.DeviceIdType.MESH,
        )
        rdma2.start()
        rdma2.wait_recv()
        out_ref[:] += recv_ref[1].astype(jnp.float32)

        rdma1.wait_send()
        rdma2.wait_send()

    out = pl.pallas_call(
        body,
        out_shape=jax.ShapeDtypeStruct((B * SQ, D_MODEL), jnp.float32),
        in_specs=[pl.BlockSpec(memory_space=pltpu.VMEM)] * 5,
        out_specs=pl.BlockSpec(memory_space=pltpu.VMEM),
        scratch_shapes=[
            pltpu.VMEM((B * SQ, D_LOC), jnp.bfloat16),
            pltpu.VMEM((2, B * SQ, D_MODEL), jnp.bfloat16),
            pltpu.VMEM((2, B * SQ, D_MODEL), jnp.bfloat16),
            pltpu.SemaphoreType.DMA((2,)),
            pltpu.SemaphoreType.DMA((2,)),
        ],
        compiler_params=pltpu.CompilerParams(collective_id=0),
    )(xf, Wq, K2, V2, Wo)
    return out.reshape(B, SQ, D_MODEL)
